# baseline (speedup 1.0000x reference)
import jax
import jax.numpy as jnp
from jax.experimental import pallas as pl
from jax.experimental.pallas import tpu as pltpu

B = 128
N = 100000
RB = 8
GR = B // RB
LA = 4
ALIGN = 99968
TAILC = N - ALIGN
CH = 1024
NFULL = ALIGN // CH
LASTW = ALIGN - NFULL * CH


def _body(act_sref, x_hbm, tail_ref, lp_ref, mode_ref, *rest):
    bufs = rest[:LA]
    sems = rest[LA:]
    i = pl.program_id(0)

    def copy(step, r):
        return pltpu.make_async_copy(
            x_hbm.at[pl.ds(step * RB, RB), pl.ds(0, ALIGN)],
            bufs[r], sems[r])

    @pl.when(i == 0)
    def _prologue():
        for s in range(LA):
            copy(s, s).start()

    for r in range(LA):
        @pl.when(jax.lax.rem(i, LA) == r)
        def _consume(r=r):
            copy(i, r).wait()
            _process(act_sref, bufs[r], tail_ref, lp_ref, mode_ref, i)
            nxt = i + LA

            @pl.when(nxt < GR)
            def _refill():
                copy(nxt, r).start()


def _process(act_sref, x_ref, tail_ref, lp_ref, mode_ref, i):
    lane = jax.lax.broadcasted_iota(jnp.int32, (RB, CH), 1)
    llane = jax.lax.broadcasted_iota(jnp.int32, (RB, LASTW), 1)
    tlane = jax.lax.broadcasted_iota(jnp.int32, (RB, TAILC), 1)

    def chunk(j):
        return x_ref[:, j * CH:(j + 1) * CH]

    last = x_ref[:, NFULL * CH:]
    tail = tail_ref[...]

    am = chunk(0)
    for j in range(1, NFULL):
        am = jnp.maximum(am, chunk(j))
    m = jnp.maximum(jnp.max(am, axis=1, keepdims=True),
                    jnp.max(last, axis=1, keepdims=True))
    m = jnp.maximum(m, jnp.max(tail, axis=1, keepdims=True))

    big = jnp.int32(2**30)
    sacc = None
    iacc = None
    for j in range(NFULL):
        xs = chunk(j)
        e = jnp.exp(xs - m)
        sacc = e if sacc is None else sacc + e
        loc = jnp.where(xs == m, j * CH + lane, big)
        iacc = loc if iacc is None else jnp.minimum(iacc, loc)
    s = jnp.sum(sacc, axis=1, keepdims=True)
    bi = jnp.min(iacc, axis=1, keepdims=True)

    s = s + jnp.sum(jnp.exp(last - m), axis=1, keepdims=True)
    lloc = jnp.where(last == m, NFULL * CH + llane, big)
    bi = jnp.minimum(bi, jnp.min(lloc, axis=1, keepdims=True))

    s = s + jnp.sum(jnp.exp(tail - m), axis=1, keepdims=True)
    tloc = jnp.where(tail == m, ALIGN + tlane, big)
    bi = jnp.minimum(bi, jnp.min(tloc, axis=1, keepdims=True))

    windows = []
    lanes = []
    acts = []
    for r in range(RB):
        a = act_sref[i * RB + r]
        start = pl.multiple_of(
            jnp.minimum(a & -128, ALIGN - 128), 128)
        windows.append(x_ref[pl.ds(r, 1), pl.ds(start, 128)])
        lanes.append(a - start)
        acts.append(a)
    ws = jnp.concatenate(windows, axis=0)
    lv = jnp.stack(lanes)[:, None]
    sel = jax.lax.broadcasted_iota(jnp.int32, (RB, 128), 1) == lv
    picked = jnp.sum(jnp.where(sel, ws, 0.0), axis=1, keepdims=True)
    av = jnp.stack(acts)[:, None]
    tsel = (ALIGN + tlane) == av
    picked = picked + jnp.sum(jnp.where(tsel, tail, 0.0),
                              axis=1, keepdims=True)

    lp_ref[...] = picked - (m + jnp.log(s))
    mode_ref[...] = bi


def _index_tail(i, _act):
    return (i, 0)


def _index_out(i, _act):
    return (i, 0)


@jax.jit
def _run(logits, actions):
    grid_spec = pltpu.PrefetchScalarGridSpec(
        num_scalar_prefetch=1,
        grid=(GR,),
        in_specs=[
            pl.BlockSpec(memory_space=pl.ANY),
            pl.BlockSpec((RB, TAILC), _index_tail),
        ],
        out_specs=[
            pl.BlockSpec((RB, 1), _index_out),
            pl.BlockSpec((RB, 1), _index_out),
        ],
        scratch_shapes=(
            [pltpu.VMEM((RB, ALIGN), jnp.float32) for _ in range(LA)]
            + [pltpu.SemaphoreType.DMA for _ in range(LA)]
        ),
    )
    lp, mode = pl.pallas_call(
        _body,
        grid_spec=grid_spec,
        out_shape=[
            jax.ShapeDtypeStruct((B, 1), jnp.float32),
            jax.ShapeDtypeStruct((B, 1), jnp.int32),
        ],
    )(actions.reshape(B), logits, logits[:, ALIGN:])
    return lp, mode


def kernel(logits, actions):
    return _run(logits, actions)
